# Initial kernel scaffold; baseline (speedup 1.0000x reference)
import jax
import jax.numpy as jnp
from jax.experimental import pallas as pl


def kernel(token_embedding, token2node):
    raise NotImplementedError("write your pallas kernel here")


# speedup vs baseline: 1.8501x; 1.8501x over previous
import jax
import jax.numpy as jnp
from jax.experimental import pallas as pl

_D_MODEL = 1024
_ENC_LEN = 4096
_NBLK = 512


def _body(t2n_ref, x_ref, rand_ref, out_ref):
    nb = pl.program_id(1)
    base = nb * _NBLK
    t2n = t2n_ref[0]
    e_l = t2n.shape[-1]
    rows = jax.lax.broadcasted_iota(jnp.int32, (_NBLK, e_l), 0) + base
    oh_bool = rows == t2n
    oh = oh_bool.astype(jnp.bfloat16)
    x = x_ref[0]
    sums = jnp.dot(oh, x, preferred_element_type=jnp.float32)
    cnt = jnp.sum(oh.astype(jnp.float32), axis=1)
    mean = sums / jnp.maximum(cnt, 1.0)[:, None]
    out_ref[0] = jnp.where(cnt[:, None] == 0.0, rand_ref[0], mean)


def _graph_embed(token_embedding, token2node, enc_len):
    b, e_l, d = token_embedding.shape
    rand = jax.random.uniform(
        jax.random.key(1), (b, enc_len + 1, d), dtype=jnp.float32)
    xb = token_embedding.astype(jnp.bfloat16)
    t2n3 = token2node.reshape(b, 1, e_l)
    n_blocks = (enc_len + 1 + _NBLK - 1) // _NBLK
    return pl.pallas_call(
        _body,
        grid=(b, n_blocks),
        in_specs=[
            pl.BlockSpec((1, 1, e_l), lambda i, j: (i, 0, 0)),
            pl.BlockSpec((1, e_l, d), lambda i, j: (i, 0, 0)),
            pl.BlockSpec((1, _NBLK, d), lambda i, j: (i, j, 0)),
        ],
        out_specs=pl.BlockSpec((1, _NBLK, d), lambda i, j: (i, j, 0)),
        out_shape=jax.ShapeDtypeStruct((b, enc_len + 1, d), jnp.float32),
    )(t2n3, xb, rand)


def kernel(token_embedding, token2node):
    return _graph_embed(token_embedding, token2node, _ENC_LEN)
